# baseline (speedup 1.0000x reference)
import jax
import jax.numpy as jnp
from jax.experimental import pallas as pl
from jax.experimental.pallas import tpu as pltpu

_BATCH = 128
_VOCAB = 200
_DIM = 128
_EDGES = tuple(range(0, 129, 8))
_NCHUNK = len(_EDGES) - 1


def _copy_kernel(w_ref, out_ref, buf_ref, sem):
    w = w_ref[...][None, :, :]
    for k in range(_NCHUNK):
        a, b = _EDGES[k], _EDGES[k + 1]
        buf_ref[pl.ds(a, b - a)] = jnp.broadcast_to(w, (b - a, _VOCAB, _DIM))
        pltpu.make_async_copy(
            buf_ref.at[pl.ds(a, b - a)],
            out_ref.at[pl.ds(a, b - a)],
            sem.at[k],
        ).start()
    for k in range(_NCHUNK):
        a, b = _EDGES[k], _EDGES[k + 1]
        pltpu.make_async_copy(
            buf_ref.at[pl.ds(a, b - a)],
            out_ref.at[pl.ds(a, b - a)],
            sem.at[k],
        ).wait()


def kernel(x, pe_weight):
    del x
    return pl.pallas_call(
        _copy_kernel,
        in_specs=[pl.BlockSpec(memory_space=pltpu.MemorySpace.VMEM)],
        out_specs=pl.BlockSpec(memory_space=pltpu.MemorySpace.HBM),
        out_shape=jax.ShapeDtypeStruct((_BATCH, _VOCAB, _DIM), jnp.float32),
        scratch_shapes=[
            pltpu.VMEM((_BATCH, _VOCAB, _DIM), jnp.float32),
            pltpu.SemaphoreType.DMA((_NCHUNK,)),
        ],
    )(pe_weight)


# speedup vs baseline: 1.0192x; 1.0192x over previous
import jax
import jax.numpy as jnp
from jax.experimental import pallas as pl
from jax.experimental.pallas import tpu as pltpu

_BATCH = 128
_VOCAB = 200
_DIM = 128
_EDGES = (0, 8, 16, 32, 64, 128)
_NCHUNK = len(_EDGES) - 1


def _copy_kernel(w_ref, out_ref, buf_ref, sem):
    w = w_ref[...][None, :, :]
    for k in range(_NCHUNK):
        a, b = _EDGES[k], _EDGES[k + 1]
        buf_ref[pl.ds(a, b - a)] = jnp.broadcast_to(w, (b - a, _VOCAB, _DIM))
        pltpu.make_async_copy(
            buf_ref.at[pl.ds(a, b - a)],
            out_ref.at[pl.ds(a, b - a)],
            sem.at[k],
        ).start()
    for k in range(_NCHUNK):
        a, b = _EDGES[k], _EDGES[k + 1]
        pltpu.make_async_copy(
            buf_ref.at[pl.ds(a, b - a)],
            out_ref.at[pl.ds(a, b - a)],
            sem.at[k],
        ).wait()


def kernel(x, pe_weight):
    del x
    return pl.pallas_call(
        _copy_kernel,
        in_specs=[pl.BlockSpec(memory_space=pltpu.MemorySpace.VMEM)],
        out_specs=pl.BlockSpec(memory_space=pltpu.MemorySpace.HBM),
        out_shape=jax.ShapeDtypeStruct((_BATCH, _VOCAB, _DIM), jnp.float32),
        scratch_shapes=[
            pltpu.VMEM((_BATCH, _VOCAB, _DIM), jnp.float32),
            pltpu.SemaphoreType.DMA((_NCHUNK,)),
        ],
    )(pe_weight)


# speedup vs baseline: 1.0232x; 1.0040x over previous
import jax
import jax.numpy as jnp
from jax.experimental import pallas as pl
from jax.experimental.pallas import tpu as pltpu

_BATCH = 128
_VOCAB = 200
_DIM = 128
_EDGES = (0, 8, 16, 32, 64)
_NFILL = len(_EDGES) - 1
_HALF = 64


def _copy_kernel(w_ref, out_ref, buf_ref, sem):
    w = w_ref[...][None, :, :]
    for k in range(_NFILL):
        a, b = _EDGES[k], _EDGES[k + 1]
        buf_ref[pl.ds(a, b - a)] = jnp.broadcast_to(w, (b - a, _VOCAB, _DIM))
        pltpu.make_async_copy(
            buf_ref.at[pl.ds(a, b - a)],
            out_ref.at[pl.ds(a, b - a)],
            sem.at[k],
        ).start()
    pltpu.make_async_copy(
        buf_ref, out_ref.at[pl.ds(_HALF, _HALF)], sem.at[_NFILL]).start()
    for k in range(_NFILL):
        a, b = _EDGES[k], _EDGES[k + 1]
        pltpu.make_async_copy(
            buf_ref.at[pl.ds(a, b - a)],
            out_ref.at[pl.ds(a, b - a)],
            sem.at[k],
        ).wait()
    pltpu.make_async_copy(
        buf_ref, out_ref.at[pl.ds(_HALF, _HALF)], sem.at[_NFILL]).wait()


def kernel(x, pe_weight):
    del x
    return pl.pallas_call(
        _copy_kernel,
        in_specs=[pl.BlockSpec(memory_space=pltpu.MemorySpace.VMEM)],
        out_specs=pl.BlockSpec(memory_space=pltpu.MemorySpace.HBM),
        out_shape=jax.ShapeDtypeStruct((_BATCH, _VOCAB, _DIM), jnp.float32),
        scratch_shapes=[
            pltpu.VMEM((_HALF, _VOCAB, _DIM), jnp.float32),
            pltpu.SemaphoreType.DMA((_NFILL + 1,)),
        ],
    )(pe_weight)


# speedup vs baseline: 1.0235x; 1.0003x over previous
import jax
import jax.numpy as jnp
from jax.experimental import pallas as pl
from jax.experimental.pallas import tpu as pltpu

_BATCH = 128
_VOCAB = 200
_DIM = 128
_EDGES = (0, 8, 16, 32)
_NFILL = len(_EDGES) - 1
_Q = 32
_NTAIL = _BATCH // _Q - 1


def _copy_kernel(w_ref, out_ref, buf_ref, sem):
    w = w_ref[...][None, :, :]
    for k in range(_NFILL):
        a, b = _EDGES[k], _EDGES[k + 1]
        buf_ref[pl.ds(a, b - a)] = jnp.broadcast_to(w, (b - a, _VOCAB, _DIM))
        pltpu.make_async_copy(
            buf_ref.at[pl.ds(a, b - a)],
            out_ref.at[pl.ds(a, b - a)],
            sem.at[k],
        ).start()
    for t in range(_NTAIL):
        pltpu.make_async_copy(
            buf_ref, out_ref.at[pl.ds(_Q * (t + 1), _Q)],
            sem.at[_NFILL + t]).start()
    for k in range(_NFILL):
        a, b = _EDGES[k], _EDGES[k + 1]
        pltpu.make_async_copy(
            buf_ref.at[pl.ds(a, b - a)],
            out_ref.at[pl.ds(a, b - a)],
            sem.at[k],
        ).wait()
    for t in range(_NTAIL):
        pltpu.make_async_copy(
            buf_ref, out_ref.at[pl.ds(_Q * (t + 1), _Q)],
            sem.at[_NFILL + t]).wait()


def kernel(x, pe_weight):
    del x
    return pl.pallas_call(
        _copy_kernel,
        in_specs=[pl.BlockSpec(memory_space=pltpu.MemorySpace.VMEM)],
        out_specs=pl.BlockSpec(memory_space=pltpu.MemorySpace.HBM),
        out_shape=jax.ShapeDtypeStruct((_BATCH, _VOCAB, _DIM), jnp.float32),
        scratch_shapes=[
            pltpu.VMEM((_Q, _VOCAB, _DIM), jnp.float32),
            pltpu.SemaphoreType.DMA((_NFILL + _NTAIL,)),
        ],
    )(pe_weight)


# speedup vs baseline: 1.0248x; 1.0012x over previous
import jax
import jax.numpy as jnp
from jax.experimental import pallas as pl
from jax.experimental.pallas import tpu as pltpu

_BATCH = 128
_VOCAB = 200
_DIM = 128
_EDGES = (0, 2, 4, 8, 16, 32)
_NFILL = len(_EDGES) - 1
_Q = 32
_NTAIL = _BATCH // _Q - 1


def _copy_kernel(w_ref, out_ref, buf_ref, sem):
    w = w_ref[...][None, :, :]
    for k in range(_NFILL):
        a, b = _EDGES[k], _EDGES[k + 1]
        buf_ref[pl.ds(a, b - a)] = jnp.broadcast_to(w, (b - a, _VOCAB, _DIM))
        pltpu.make_async_copy(
            buf_ref.at[pl.ds(a, b - a)],
            out_ref.at[pl.ds(a, b - a)],
            sem.at[k],
        ).start()
    for t in range(_NTAIL):
        pltpu.make_async_copy(
            buf_ref, out_ref.at[pl.ds(_Q * (t + 1), _Q)],
            sem.at[_NFILL + t]).start()
    for k in range(_NFILL):
        a, b = _EDGES[k], _EDGES[k + 1]
        pltpu.make_async_copy(
            buf_ref.at[pl.ds(a, b - a)],
            out_ref.at[pl.ds(a, b - a)],
            sem.at[k],
        ).wait()
    for t in range(_NTAIL):
        pltpu.make_async_copy(
            buf_ref, out_ref.at[pl.ds(_Q * (t + 1), _Q)],
            sem.at[_NFILL + t]).wait()


def kernel(x, pe_weight):
    del x
    return pl.pallas_call(
        _copy_kernel,
        in_specs=[pl.BlockSpec(memory_space=pltpu.MemorySpace.VMEM)],
        out_specs=pl.BlockSpec(memory_space=pltpu.MemorySpace.HBM),
        out_shape=jax.ShapeDtypeStruct((_BATCH, _VOCAB, _DIM), jnp.float32),
        scratch_shapes=[
            pltpu.VMEM((_Q, _VOCAB, _DIM), jnp.float32),
            pltpu.SemaphoreType.DMA((_NFILL + _NTAIL,)),
        ],
    )(pe_weight)
